# Initial kernel scaffold; baseline (speedup 1.0000x reference)
import jax
import jax.numpy as jnp
from jax.experimental import pallas as pl


def kernel(x, edge_index, Wl1, bl1, Wr1, Wl2, bl2, Wr2, Wf, bf):
    raise NotImplementedError("write your pallas kernel here")


# speedup vs baseline: 4.6628x; 4.6628x over previous
import functools

import jax
import jax.numpy as jnp
from jax import lax
from jax.experimental import pallas as pl
from jax.experimental.pallas import tpu as pltpu
from jax.experimental.pallas import tpu_sc as plsc

N_NODES = 10000
N_EDGES = 320000
D = 128

NC = 2
NS = 16
NW = NC * NS
EDGES_PER_TILE = N_EDGES // NW
CHUNK = 80
NCHUNKS = EDGES_PER_TILE // CHUNK
N_PAD = 10240
ROWS_PER_TILE = N_PAD // NS


def _sc_mesh():
  return plsc.VectorSubcoreMesh(
      core_axis_name="c", subcore_axis_name="s", num_cores=NC,
      num_subcores=NS)


@functools.cache
def _make_sc_agg():
  def body(x_hbm, src_hbm, dst_hbm, zero_hbm, out_hbm,
           agg_sh, src_v, dst_v, rows_v, sem):
    cid = lax.axis_index("c")
    sid = lax.axis_index("s")
    wid = cid * NS + sid
    pltpu.sync_copy(
        zero_hbm, agg_sh.at[pl.ds(sid * ROWS_PER_TILE, ROWS_PER_TILE)])
    plsc.subcore_barrier()

    def chunk(j, carry):
      base = wid * EDGES_PER_TILE + j * CHUNK
      pltpu.sync_copy(src_hbm.at[pl.ds(base, CHUNK)], src_v)
      pltpu.sync_copy(dst_hbm.at[pl.ds(base, CHUNK)], dst_v)
      pltpu.async_copy(x_hbm.at[src_v], rows_v, sem).wait()
      pltpu.sync_copy(rows_v, agg_sh.at[dst_v], add=True)
      return carry

    lax.fori_loop(0, NCHUNKS, chunk, 0)
    plsc.subcore_barrier()
    pltpu.sync_copy(
        agg_sh.at[pl.ds(sid * ROWS_PER_TILE, ROWS_PER_TILE)],
        out_hbm.at[cid, pl.ds(sid * ROWS_PER_TILE, ROWS_PER_TILE)])

  return pl.kernel(
      body,
      out_type=jax.ShapeDtypeStruct((NC, N_PAD, D), jnp.float32),
      mesh=_sc_mesh(),
      scratch_types=[
          pltpu.VMEM_SHARED((N_PAD, D), jnp.float32),
          pltpu.VMEM((CHUNK,), jnp.int32),
          pltpu.VMEM((CHUNK,), jnp.int32),
          pltpu.VMEM((CHUNK, D), jnp.float32),
          pltpu.SemaphoreType.DMA,
      ],
  )


@functools.cache
def _make_sc_deg():
  def body(dst_hbm, ones_hbm, zero_hbm, out_hbm, deg_sh, dst_v, ones_v):
    cid = lax.axis_index("c")
    sid = lax.axis_index("s")
    wid = cid * NS + sid
    pltpu.sync_copy(
        zero_hbm, deg_sh.at[pl.ds(sid * ROWS_PER_TILE, ROWS_PER_TILE)])
    pltpu.sync_copy(ones_hbm, ones_v)
    plsc.subcore_barrier()

    def chunk(j, carry):
      base = wid * EDGES_PER_TILE + j * CHUNK
      pltpu.sync_copy(dst_hbm.at[pl.ds(base, CHUNK)], dst_v)
      pltpu.sync_copy(ones_v, deg_sh.at[dst_v], add=True)
      return carry

    lax.fori_loop(0, NCHUNKS, chunk, 0)
    plsc.subcore_barrier()
    pltpu.sync_copy(
        deg_sh.at[pl.ds(sid * ROWS_PER_TILE, ROWS_PER_TILE)],
        out_hbm.at[cid, pl.ds(sid * ROWS_PER_TILE, ROWS_PER_TILE)])

  return pl.kernel(
      body,
      out_type=jax.ShapeDtypeStruct((NC, N_PAD, D), jnp.float32),
      mesh=_sc_mesh(),
      scratch_types=[
          pltpu.VMEM_SHARED((N_PAD, D), jnp.float32),
          pltpu.VMEM((CHUNK,), jnp.int32),
          pltpu.VMEM((CHUNK, D), jnp.float32),
      ],
  )


ROW_BLK = 1000
GRID = N_NODES // ROW_BLK


def _tc1_body(p0, p1, d0, d1, x, wl, wr, b, h, degc):
  deg = jnp.maximum(d0[:, :1] + d1[:, :1], 1.0)
  agg = p0[...] + p1[...]
  m = lax.dot_general(agg, wl[...], (((1,), (1,)), ((), ())),
                      preferred_element_type=jnp.float32) / deg
  r = lax.dot_general(x[...], wr[...], (((1,), (1,)), ((), ())),
                      preferred_element_type=jnp.float32)
  h[...] = jnp.maximum(m + r + b[...], 0.0)
  degc[...] = deg


_tc1 = pl.pallas_call(
    _tc1_body,
    grid=(GRID,),
    in_specs=[
        pl.BlockSpec((ROW_BLK, D), lambda i: (i, 0)),
        pl.BlockSpec((ROW_BLK, D), lambda i: (i, 0)),
        pl.BlockSpec((ROW_BLK, D), lambda i: (i, 0)),
        pl.BlockSpec((ROW_BLK, D), lambda i: (i, 0)),
        pl.BlockSpec((ROW_BLK, D), lambda i: (i, 0)),
        pl.BlockSpec((D, D), lambda i: (0, 0)),
        pl.BlockSpec((D, D), lambda i: (0, 0)),
        pl.BlockSpec((1, D), lambda i: (0, 0)),
    ],
    out_specs=[
        pl.BlockSpec((ROW_BLK, D), lambda i: (i, 0)),
        pl.BlockSpec((ROW_BLK, 1), lambda i: (i, 0)),
    ],
    out_shape=[
        jax.ShapeDtypeStruct((N_NODES, D), jnp.float32),
        jax.ShapeDtypeStruct((N_NODES, 1), jnp.float32),
    ],
)


def _tc2_body(q0, q1, h, degc, wl, wr, b, wf, bf, out):
  agg = q0[...] + q1[...]
  m = lax.dot_general(agg, wl[...], (((1,), (1,)), ((), ())),
                      preferred_element_type=jnp.float32) / degc[...]
  r = lax.dot_general(h[...], wr[...], (((1,), (1,)), ((), ())),
                      preferred_element_type=jnp.float32)
  h2 = jnp.maximum(m + r + b[...], 0.0)
  out[...] = lax.dot_general(h2, wf[...], (((1,), (1,)), ((), ())),
                             preferred_element_type=jnp.float32) + bf[...]


_tc2 = pl.pallas_call(
    _tc2_body,
    grid=(GRID,),
    in_specs=[
        pl.BlockSpec((ROW_BLK, D), lambda i: (i, 0)),
        pl.BlockSpec((ROW_BLK, D), lambda i: (i, 0)),
        pl.BlockSpec((ROW_BLK, D), lambda i: (i, 0)),
        pl.BlockSpec((ROW_BLK, 1), lambda i: (i, 0)),
        pl.BlockSpec((D, D), lambda i: (0, 0)),
        pl.BlockSpec((D, D), lambda i: (0, 0)),
        pl.BlockSpec((1, D), lambda i: (0, 0)),
        pl.BlockSpec((D, D), lambda i: (0, 0)),
        pl.BlockSpec((1, D), lambda i: (0, 0)),
    ],
    out_specs=pl.BlockSpec((ROW_BLK, D), lambda i: (i, 0)),
    out_shape=jax.ShapeDtypeStruct((N_NODES, D), jnp.float32),
)


@jax.jit
def kernel(x, edge_index, Wl1, bl1, Wr1, Wl2, bl2, Wr2, Wf, bf):
  src = edge_index[0].astype(jnp.int32)
  dst = edge_index[1].astype(jnp.int32)
  zero = jnp.zeros((ROWS_PER_TILE, D), jnp.float32)
  ones = jnp.ones((CHUNK, D), jnp.float32)

  dp = _make_sc_deg()(dst, ones, zero)
  p = _make_sc_agg()(x, src, dst, zero)
  h, degc = _tc1(p[0], p[1], dp[0], dp[1], x, Wl1, Wr1, bl1.reshape(1, D))
  q = _make_sc_agg()(h, src, dst, zero)
  out = _tc2(q[0], q[1], h, degc, Wl2, Wr2, bl2.reshape(1, D),
             Wf, bf.reshape(1, D))
  return out


# speedup vs baseline: 9.6847x; 2.0770x over previous
import functools

import jax
import jax.numpy as jnp
from jax import lax
from jax.experimental import pallas as pl
from jax.experimental.pallas import tpu as pltpu
from jax.experimental.pallas import tpu_sc as plsc

N_NODES = 10000
N_EDGES = 320000
D = 128

NC = 2
NS = 16
NW = NC * NS
EDGES_PER_TILE = N_EDGES // NW
CHUNK = 80
NCHUNKS = EDGES_PER_TILE // CHUNK
N_PAD = 10240
ROWS_PER_TILE = N_PAD // NS


def _sc_mesh():
  return plsc.VectorSubcoreMesh(
      core_axis_name="c", subcore_axis_name="s", num_cores=NC,
      num_subcores=NS)


@functools.cache
def _make_sc_agg():
  def body(x_hbm, src_hbm, dst_hbm, zero_hbm, out_hbm,
           agg_sh, src_v, dst_v, rows0, rows1, sem0, sem1):
    cid = lax.axis_index("c")
    sid = lax.axis_index("s")
    wid = cid * NS + sid
    pltpu.sync_copy(
        zero_hbm, agg_sh.at[pl.ds(sid * ROWS_PER_TILE, ROWS_PER_TILE)])
    pltpu.sync_copy(src_hbm.at[wid], src_v)
    pltpu.sync_copy(dst_hbm.at[wid], dst_v)
    plsc.subcore_barrier()

    def gather(j, rows, sem):
      pltpu.async_copy(
          x_hbm.at[src_v.at[pl.ds(j * CHUNK, CHUNK)]], rows, sem)

    def wait(rows, sem):
      pltpu.make_async_copy(x_hbm.at[pl.ds(0, CHUNK)], rows, sem).wait()

    gather(0, rows0, sem0)
    gather(1, rows1, sem1)

    def pair(t, carry):
      j = 2 * t
      wait(rows0, sem0)
      pltpu.sync_copy(rows0, agg_sh.at[dst_v.at[j]], add=True)
      gather(j + 2, rows0, sem0)
      wait(rows1, sem1)
      pltpu.sync_copy(rows1, agg_sh.at[dst_v.at[j + 1]], add=True)

      @pl.when(t < (NCHUNKS - 1) // 2 - 1)
      def _():
        gather(j + 3, rows1, sem1)
      return carry

    lax.fori_loop(0, (NCHUNKS - 1) // 2, pair, 0)
    wait(rows0, sem0)
    pltpu.sync_copy(rows0, agg_sh.at[dst_v.at[NCHUNKS - 1]], add=True)
    plsc.subcore_barrier()
    pltpu.sync_copy(
        agg_sh.at[pl.ds(sid * ROWS_PER_TILE, ROWS_PER_TILE)],
        out_hbm.at[cid, pl.ds(sid * ROWS_PER_TILE, ROWS_PER_TILE)])

  return pl.kernel(
      body,
      out_type=jax.ShapeDtypeStruct((NC, N_PAD, D), jnp.float32),
      mesh=_sc_mesh(),
      scratch_types=[
          pltpu.VMEM_SHARED((N_PAD, D), jnp.float32),
          pltpu.VMEM((EDGES_PER_TILE,), jnp.int32),
          pltpu.VMEM((NCHUNKS, CHUNK), jnp.int32),
          pltpu.VMEM((CHUNK, D), jnp.float32),
          pltpu.VMEM((CHUNK, D), jnp.float32),
          pltpu.SemaphoreType.DMA,
          pltpu.SemaphoreType.DMA,
      ],
  )


@functools.cache
def _make_sc_deg():
  def body(dst_hbm, ones_hbm, zero_hbm, out_hbm, deg_sh, dst_v, ones_v):
    cid = lax.axis_index("c")
    sid = lax.axis_index("s")
    wid = cid * NS + sid
    pltpu.sync_copy(
        zero_hbm, deg_sh.at[pl.ds(sid * ROWS_PER_TILE, ROWS_PER_TILE)])
    pltpu.sync_copy(ones_hbm, ones_v)
    pltpu.sync_copy(dst_hbm.at[wid], dst_v)
    plsc.subcore_barrier()

    def chunk(j, carry):
      pltpu.sync_copy(ones_v, deg_sh.at[dst_v.at[j]], add=True)
      return carry

    lax.fori_loop(0, NCHUNKS, chunk, 0)
    plsc.subcore_barrier()
    pltpu.sync_copy(
        deg_sh.at[pl.ds(sid * ROWS_PER_TILE, ROWS_PER_TILE)],
        out_hbm.at[cid, pl.ds(sid * ROWS_PER_TILE, ROWS_PER_TILE)])

  return pl.kernel(
      body,
      out_type=jax.ShapeDtypeStruct((NC, N_PAD, D), jnp.float32),
      mesh=_sc_mesh(),
      scratch_types=[
          pltpu.VMEM_SHARED((N_PAD, D), jnp.float32),
          pltpu.VMEM((NCHUNKS, CHUNK), jnp.int32),
          pltpu.VMEM((CHUNK, D), jnp.float32),
      ],
  )


ROW_BLK = 1000
GRID = N_NODES // ROW_BLK


def _tc1_body(p0, p1, d0, d1, x, wl, wr, b, h, degc):
  deg = jnp.maximum(d0[:, :1] + d1[:, :1], 1.0)
  agg = p0[...] + p1[...]
  m = lax.dot_general(agg, wl[...], (((1,), (1,)), ((), ())),
                      preferred_element_type=jnp.float32) / deg
  r = lax.dot_general(x[...], wr[...], (((1,), (1,)), ((), ())),
                      preferred_element_type=jnp.float32)
  h[...] = jnp.maximum(m + r + b[...], 0.0)
  degc[...] = deg


_tc1 = pl.pallas_call(
    _tc1_body,
    grid=(GRID,),
    in_specs=[
        pl.BlockSpec((ROW_BLK, D), lambda i: (i, 0)),
        pl.BlockSpec((ROW_BLK, D), lambda i: (i, 0)),
        pl.BlockSpec((ROW_BLK, D), lambda i: (i, 0)),
        pl.BlockSpec((ROW_BLK, D), lambda i: (i, 0)),
        pl.BlockSpec((ROW_BLK, D), lambda i: (i, 0)),
        pl.BlockSpec((D, D), lambda i: (0, 0)),
        pl.BlockSpec((D, D), lambda i: (0, 0)),
        pl.BlockSpec((1, D), lambda i: (0, 0)),
    ],
    out_specs=[
        pl.BlockSpec((ROW_BLK, D), lambda i: (i, 0)),
        pl.BlockSpec((ROW_BLK, 1), lambda i: (i, 0)),
    ],
    out_shape=[
        jax.ShapeDtypeStruct((N_NODES, D), jnp.float32),
        jax.ShapeDtypeStruct((N_NODES, 1), jnp.float32),
    ],
)


def _tc2_body(q0, q1, h, degc, wl, wr, b, wf, bf, out):
  agg = q0[...] + q1[...]
  m = lax.dot_general(agg, wl[...], (((1,), (1,)), ((), ())),
                      preferred_element_type=jnp.float32) / degc[...]
  r = lax.dot_general(h[...], wr[...], (((1,), (1,)), ((), ())),
                      preferred_element_type=jnp.float32)
  h2 = jnp.maximum(m + r + b[...], 0.0)
  out[...] = lax.dot_general(h2, wf[...], (((1,), (1,)), ((), ())),
                             preferred_element_type=jnp.float32) + bf[...]


_tc2 = pl.pallas_call(
    _tc2_body,
    grid=(GRID,),
    in_specs=[
        pl.BlockSpec((ROW_BLK, D), lambda i: (i, 0)),
        pl.BlockSpec((ROW_BLK, D), lambda i: (i, 0)),
        pl.BlockSpec((ROW_BLK, D), lambda i: (i, 0)),
        pl.BlockSpec((ROW_BLK, 1), lambda i: (i, 0)),
        pl.BlockSpec((D, D), lambda i: (0, 0)),
        pl.BlockSpec((D, D), lambda i: (0, 0)),
        pl.BlockSpec((1, D), lambda i: (0, 0)),
        pl.BlockSpec((D, D), lambda i: (0, 0)),
        pl.BlockSpec((1, D), lambda i: (0, 0)),
    ],
    out_specs=pl.BlockSpec((ROW_BLK, D), lambda i: (i, 0)),
    out_shape=jax.ShapeDtypeStruct((N_NODES, D), jnp.float32),
)


@jax.jit
def kernel(x, edge_index, Wl1, bl1, Wr1, Wl2, bl2, Wr2, Wf, bf):
  src = edge_index[0].astype(jnp.int32).reshape(NW, EDGES_PER_TILE)
  dst = edge_index[1].astype(jnp.int32).reshape(NW, NCHUNKS, CHUNK)
  zero = jnp.zeros((ROWS_PER_TILE, D), jnp.float32)
  ones = jnp.ones((CHUNK, D), jnp.float32)

  dp = _make_sc_deg()(dst, ones, zero)
  p = _make_sc_agg()(x, src, dst, zero)
  h, degc = _tc1(p[0], p[1], dp[0], dp[1], x, Wl1, Wr1, bl1.reshape(1, D))
  q = _make_sc_agg()(h, src, dst, zero)
  out = _tc2(q[0], q[1], h, degc, Wl2, Wr2, bl2.reshape(1, D),
             Wf, bf.reshape(1, D))
  return out
